# baseline (speedup 1.0000x reference)
import functools

import jax
import jax.numpy as jnp
from jax import lax
from jax.experimental import pallas as pl

HEADS = 2
FEAT = 64
ORDER = 180

_BR = 256


def _mm_body(x_ref, w_ref, b_ref, o_ref, *, relu):
    acc = lax.dot_general(
        x_ref[...], w_ref[...], (((1,), (0,)), ((), ())),
        preferred_element_type=jnp.float32,
    ) + b_ref[...]
    if relu:
        acc = jnp.maximum(acc, 0.0)
    o_ref[...] = acc


def _pad_to(a, axis, mult):
    n = a.shape[axis]
    p = (-n) % mult
    if p == 0:
        return a
    pads = [(0, 0)] * a.ndim
    pads[axis] = (0, p)
    return jnp.pad(a, pads)


def _dense(x, w, b=None, relu=False):
    m, k = x.shape
    _, n = w.shape
    if b is None:
        b = jnp.zeros((n,), jnp.float32)
    xp = _pad_to(_pad_to(x, 1, 128), 0, _BR)
    wp = _pad_to(_pad_to(w, 0, 128), 1, 128)
    bp = _pad_to(b[None, :], 1, 128)
    mp, kp = xp.shape
    np_ = wp.shape[1]
    out = pl.pallas_call(
        functools.partial(_mm_body, relu=relu),
        grid=(mp // _BR,),
        in_specs=[
            pl.BlockSpec((_BR, kp), lambda i: (i, 0)),
            pl.BlockSpec((kp, np_), lambda i: (0, 0)),
            pl.BlockSpec((1, np_), lambda i: (0, 0)),
        ],
        out_specs=pl.BlockSpec((_BR, np_), lambda i: (i, 0)),
        out_shape=jax.ShapeDtypeStruct((mp, np_), jnp.float32),
    )(xp, wp, bp)
    return out[:m, :n]


def _acat(a_s, a_d):
    z = jnp.zeros((FEAT,), jnp.float32)
    c0 = jnp.concatenate([a_s[0], z])
    c1 = jnp.concatenate([z, a_s[1]])
    c2 = jnp.concatenate([a_d[0], z])
    c3 = jnp.concatenate([z, a_d[1]])
    return jnp.stack([c0, c1, c2, c3], axis=1)


def _gat(x, src, dst, W, a_s, a_d, b):
    n = x.shape[0]
    wal = jnp.concatenate([W, W @ _acat(a_s, a_d)], axis=1)
    hal = _dense(x, wal)
    h = hal[:, : HEADS * FEAT].reshape(n, HEADS, FEAT)
    al_s = hal[:, HEADS * FEAT : HEADS * FEAT + HEADS]
    al_d = hal[:, HEADS * FEAT + HEADS :]
    e = jax.nn.leaky_relu(al_s[src] + al_d[dst], 0.2)
    m = jax.ops.segment_max(e, dst, num_segments=n, indices_are_sorted=True)
    m = jnp.where(jnp.isfinite(m), m, 0.0)
    ex = jnp.exp(e - m[dst])
    den = jax.ops.segment_sum(ex, dst, num_segments=n, indices_are_sorted=True)
    alpha = ex / (den[dst] + 1e-16)
    out = jax.ops.segment_sum(h[src] * alpha[:, :, None], dst, num_segments=n,
                              indices_are_sorted=True)
    return out.reshape(n, HEADS * FEAT) + b


def kernel(z, condition, edge_index, batch, W_init, b_init, W1, a1s, a1d, b1,
           W2, a2s, a2d, b2, W3, a3s, a3d, b3, Wp, bp, Wfp, bfp, Ws, bs,
           Wfs, bfs, Wt, bt, Wft, bft):
    n = batch.shape[0]
    zc = _dense(jnp.concatenate([z, condition], axis=1), W_init, b_init,
                relu=True)
    zn = zc[batch]
    starts = jnp.searchsorted(batch, batch, side='left')
    order = jnp.clip(jnp.arange(n) - starts, 0, ORDER - 1)
    pos = jax.nn.one_hot(order, ORDER, dtype=jnp.float32)
    x0 = jnp.concatenate([zn, pos], axis=1)
    loops = jnp.arange(n, dtype=edge_index.dtype)
    ei = jnp.concatenate([edge_index, jnp.stack([loops, loops])], axis=1)
    perm = jnp.argsort(ei[1])
    src, dst = ei[0][perm], ei[1][perm]
    x1 = jax.nn.relu(_gat(x0, src, dst, W1, a1s, a1d, b1))
    x2 = jax.nn.relu(_gat(x1, src, dst, W2, a2s, a2d, b2))
    x3 = jax.nn.relu(_gat(x2, src, dst, W3, a3s, a3d, b3))
    out_pos = _dense(_dense(x3, Wp, bp, relu=True), Wfp, bfp)
    out_size = _dense(_dense(x3, Ws, bs, relu=True), Wfs, bfs)
    out_theta = _dense(_dense(x3, Wt, bt, relu=True), Wft, bft)
    return (out_pos, out_size, out_theta)


# speedup vs baseline: 1.1265x; 1.1265x over previous
import functools

import jax
import jax.numpy as jnp
from jax import lax
from jax.experimental import pallas as pl

HEADS = 2
FEAT = 64
ORDER = 180

_BR = 256


def _mm_body(x_ref, w_ref, b_ref, o_ref, *, relu):
    acc = lax.dot_general(
        x_ref[...], w_ref[...], (((1,), (0,)), ((), ())),
        preferred_element_type=jnp.float32,
    ) + b_ref[...]
    if relu:
        acc = jnp.maximum(acc, 0.0)
    o_ref[...] = acc


def _pad_to(a, axis, mult):
    n = a.shape[axis]
    p = (-n) % mult
    if p == 0:
        return a
    pads = [(0, 0)] * a.ndim
    pads[axis] = (0, p)
    return jnp.pad(a, pads)


def _dense(x, w, b=None, relu=False):
    m, k = x.shape
    _, n = w.shape
    if b is None:
        b = jnp.zeros((n,), jnp.float32)
    xp = _pad_to(_pad_to(x, 1, 128), 0, _BR)
    wp = _pad_to(_pad_to(w, 0, 128), 1, 128)
    bp = _pad_to(b[None, :], 1, 128)
    mp, kp = xp.shape
    np_ = wp.shape[1]
    out = pl.pallas_call(
        functools.partial(_mm_body, relu=relu),
        grid=(mp // _BR,),
        in_specs=[
            pl.BlockSpec((_BR, kp), lambda i: (i, 0)),
            pl.BlockSpec((kp, np_), lambda i: (0, 0)),
            pl.BlockSpec((1, np_), lambda i: (0, 0)),
        ],
        out_specs=pl.BlockSpec((_BR, np_), lambda i: (i, 0)),
        out_shape=jax.ShapeDtypeStruct((mp, np_), jnp.float32),
    )(xp, wp, bp)
    return out[:m, :n]


def _acat(a_s, a_d):
    z = jnp.zeros((FEAT,), jnp.float32)
    c0 = jnp.concatenate([a_s[0], z])
    c1 = jnp.concatenate([z, a_s[1]])
    c2 = jnp.concatenate([a_d[0], z])
    c3 = jnp.concatenate([z, a_d[1]])
    return jnp.stack([c0, c1, c2, c3], axis=1)


def _gat(x, src, dst, W, a_s, a_d, b):
    n = x.shape[0]
    wal = jnp.concatenate([W, W @ _acat(a_s, a_d)], axis=1)
    hal = _dense(x, wal)
    h = hal[:, : HEADS * FEAT].reshape(n, HEADS, FEAT)
    al_s = hal[:, HEADS * FEAT : HEADS * FEAT + HEADS]
    al_d = hal[:, HEADS * FEAT + HEADS :]
    e = jax.nn.leaky_relu(al_s[src] + al_d[dst], 0.2)
    m = jax.ops.segment_max(e, dst, num_segments=n)
    m = jnp.where(jnp.isfinite(m), m, 0.0)
    ex = jnp.exp(e - m[dst])
    den = jax.ops.segment_sum(ex, dst, num_segments=n)
    alpha = ex / (den[dst] + 1e-16)
    out = jax.ops.segment_sum(h[src] * alpha[:, :, None], dst, num_segments=n)
    return out.reshape(n, HEADS * FEAT) + b


def kernel(z, condition, edge_index, batch, W_init, b_init, W1, a1s, a1d, b1,
           W2, a2s, a2d, b2, W3, a3s, a3d, b3, Wp, bp, Wfp, bfp, Ws, bs,
           Wfs, bfs, Wt, bt, Wft, bft):
    n = batch.shape[0]
    zc = _dense(jnp.concatenate([z, condition], axis=1), W_init, b_init,
                relu=True)
    zn = zc[batch]
    starts = jnp.searchsorted(batch, batch, side='left')
    order = jnp.clip(jnp.arange(n) - starts, 0, ORDER - 1)
    pos = jax.nn.one_hot(order, ORDER, dtype=jnp.float32)
    x0 = jnp.concatenate([zn, pos], axis=1)
    loops = jnp.arange(n, dtype=edge_index.dtype)
    ei = jnp.concatenate([edge_index, jnp.stack([loops, loops])], axis=1)
    src, dst = ei[0], ei[1]
    x1 = jax.nn.relu(_gat(x0, src, dst, W1, a1s, a1d, b1))
    x2 = jax.nn.relu(_gat(x1, src, dst, W2, a2s, a2d, b2))
    x3 = jax.nn.relu(_gat(x2, src, dst, W3, a3s, a3d, b3))
    out_pos = _dense(_dense(x3, Wp, bp, relu=True), Wfp, bfp)
    out_size = _dense(_dense(x3, Ws, bs, relu=True), Wfs, bfs)
    out_theta = _dense(_dense(x3, Wt, bt, relu=True), Wft, bft)
    return (out_pos, out_size, out_theta)
